# baseline (speedup 1.0000x reference)
import functools

import jax
import jax.numpy as jnp
from jax.experimental import pallas as pl

_T = 64
_T2 = 128
_C = 128
_G = 100


def _body(x_ref, wq_ref, wk_ref, wv_ref, o_ref, *, g):
    xb = x_ref[...]
    x2 = xb.reshape(g * _T2, _C)
    q = jnp.dot(x2, wq_ref[...], preferred_element_type=jnp.float32)
    k = jnp.dot(x2, wk_ref[...], preferred_element_type=jnp.float32)
    v = jnp.dot(x2, wv_ref[...], preferred_element_type=jnp.float32)
    q = q.reshape(g, _T2, _C)
    k = k.reshape(g, _T2, _C)
    v = v.reshape(g, _T2, _C)

    et = jax.lax.dot_general(
        k, q, (((2,), (2,)), ((0,), (0,))),
        preferred_element_type=jnp.float32)

    jj = jax.lax.broadcasted_iota(jnp.int32, (_T2, _T2), 0)
    tt = jax.lax.broadcasted_iota(jnp.int32, (_T2, _T2), 1)
    same = (jj < _T) == (tt < _T)
    neg_inf = jnp.float32(float("-inf"))
    et = jnp.where(same[None], et, neg_inf)

    m1 = jnp.max(et, axis=1, keepdims=True)
    e1 = jnp.where(et == m1, neg_inf, et)
    m2 = jnp.max(e1, axis=1, keepdims=True)
    e2 = jnp.where(e1 == m2, neg_inf, e1)
    m3 = jnp.max(e2, axis=1, keepdims=True)
    score = jnp.where(et >= m3, jax.nn.relu(et), jnp.float32(0.0))

    out = jax.lax.dot_general(
        score, v, (((1,), (1,)), ((0,), (0,))),
        preferred_element_type=jnp.float32)
    o_ref[...] = out


@jax.jit
def kernel(x, Wq, Wk, Wv):
    B, N, T, C = x.shape
    S = B * N
    g = _G
    xs = x.reshape(S // 2, 2 * T, C)
    out = pl.pallas_call(
        functools.partial(_body, g=g),
        grid=(S // 2 // g,),
        in_specs=[
            pl.BlockSpec((g, 2 * T, C), lambda i: (i, 0, 0)),
            pl.BlockSpec((C, C), lambda i: (0, 0)),
            pl.BlockSpec((C, C), lambda i: (0, 0)),
            pl.BlockSpec((C, C), lambda i: (0, 0)),
        ],
        out_specs=pl.BlockSpec((g, 2 * T, C), lambda i: (i, 0, 0)),
        out_shape=jax.ShapeDtypeStruct((S // 2, 2 * T, C), jnp.float32),
    )(xs, Wq, Wk, Wv)
    return out.reshape(B, N, T, C)


# speedup vs baseline: 1.0570x; 1.0570x over previous
import functools

import jax
import jax.numpy as jnp
from jax.experimental import pallas as pl
from jax.experimental.pallas import tpu as pltpu

_T = 64
_C = 128
_K = 3


def _body(x_ref, wq_ref, wk_ref, wv_ref, o_ref, *, g):
    xb = x_ref[...]
    x2 = xb.reshape(g * _T, _C)
    wq = wq_ref[...]
    wk = wk_ref[...]
    wv = wv_ref[...]
    q = jnp.dot(x2, wq, preferred_element_type=jnp.float32).reshape(g, _T, _C)
    k = jnp.dot(x2, wk, preferred_element_type=jnp.float32).reshape(g, _T, _C)
    v = jnp.dot(x2, wv, preferred_element_type=jnp.float32).reshape(g, _T, _C)

    et = jax.lax.dot_general(
        k, q, (((2,), (2,)), ((0,), (0,))),
        preferred_element_type=jnp.float32)

    neg_inf = jnp.float32(float("-inf"))
    m1 = jnp.max(et, axis=1, keepdims=True)
    e1 = jnp.where(et == m1, neg_inf, et)
    m2 = jnp.max(e1, axis=1, keepdims=True)
    e2 = jnp.where(e1 == m2, neg_inf, e1)
    m3 = jnp.max(e2, axis=1, keepdims=True)
    score = jnp.where(et >= m3, jax.nn.relu(et), jnp.float32(0.0))

    out = jax.lax.dot_general(
        score, v, (((1,), (1,)), ((0,), (0,))),
        preferred_element_type=jnp.float32)
    o_ref[...] = out


@jax.jit
def kernel(x, Wq, Wk, Wv):
    B, N, T, C = x.shape
    S = B * N
    g = 200
    xs = x.reshape(S, T, C)
    out = pl.pallas_call(
        functools.partial(_body, g=g),
        grid=(S // g,),
        in_specs=[
            pl.BlockSpec((g, T, C), lambda i: (i, 0, 0)),
            pl.BlockSpec((C, C), lambda i: (0, 0)),
            pl.BlockSpec((C, C), lambda i: (0, 0)),
            pl.BlockSpec((C, C), lambda i: (0, 0)),
        ],
        out_specs=pl.BlockSpec((g, T, C), lambda i: (i, 0, 0)),
        out_shape=jax.ShapeDtypeStruct((S, T, C), jnp.float32),
        compiler_params=pltpu.CompilerParams(
            dimension_semantics=("parallel",),
            vmem_limit_bytes=128 * 1024 * 1024,
        ),
    )(xs, Wq, Wk, Wv)
    return out.reshape(B, N, T, C)
